# Initial kernel scaffold; baseline (speedup 1.0000x reference)
import jax
import jax.numpy as jnp
from jax.experimental import pallas as pl


def kernel(x, W):
    raise NotImplementedError("write your pallas kernel here")


# speedup vs baseline: 43.8319x; 43.8319x over previous
import jax
import jax.numpy as jnp
from jax import lax
from jax.experimental import pallas as pl
from jax.experimental.pallas import tpu as pltpu
from jax.experimental.pallas import tpu_sc as plsc

N, K, D = 32768, 1024, 64
ROWS_PER_BLOCK = 1024
NUM_BLOCKS = N // ROWS_PER_BLOCK
NW = 32
B_PER_W = N // NW
IDX_CHUNK = 128
CHUNKS = B_PER_W // IDX_CHUNK


def _argmax_body(x_ref, idx_ref):
    xb = x_ref[...]
    m = jnp.max(xb, axis=-1, keepdims=True)
    col = lax.broadcasted_iota(jnp.int32, xb.shape, 1)
    cand = jnp.where(xb == m, col, K)
    idx_ref[0, 0, :] = jnp.min(cand, axis=-1)


def _tc_argmax(x):
    return pl.pallas_call(
        _argmax_body,
        grid=(NUM_BLOCKS,),
        in_specs=[pl.BlockSpec((ROWS_PER_BLOCK, K), lambda i: (i, 0))],
        out_specs=pl.BlockSpec((1, 1, ROWS_PER_BLOCK), lambda i: (i, 0, 0)),
        out_shape=jax.ShapeDtypeStruct((NUM_BLOCKS, 1, ROWS_PER_BLOCK), jnp.int32),
    )(x)


def _sc_gather_body(table_hbm, idx_hbm, out_hbm, idx_v, rows_v, sem):
    wid = lax.axis_index("s") * 2 + lax.axis_index("c")
    pltpu.sync_copy(idx_hbm.at[wid], idx_v)
    for j in range(CHUNKS):
        pltpu.async_copy(
            table_hbm.at[idx_v.at[j]],
            rows_v.at[pl.ds(j * IDX_CHUNK, IDX_CHUNK)],
            sem,
        ).wait()
    pltpu.sync_copy(rows_v, out_hbm.at[pl.ds(wid * B_PER_W, B_PER_W)])


def _sc_gather(table, idx3):
    mesh = plsc.VectorSubcoreMesh(core_axis_name="c", subcore_axis_name="s")
    run = pl.kernel(
        _sc_gather_body,
        out_type=jax.ShapeDtypeStruct((N, D), jnp.float32),
        mesh=mesh,
        scratch_types=[
            pltpu.VMEM((CHUNKS, IDX_CHUNK), jnp.int32),
            pltpu.VMEM((B_PER_W, D), jnp.float32),
            pltpu.SemaphoreType.DMA,
        ],
        compiler_params=pltpu.CompilerParams(use_tc_tiling_on_sc=False),
    )
    return run(table, idx3)


def kernel(x, W):
    idx = _tc_argmax(x).reshape(NW, CHUNKS, IDX_CHUNK)
    table = jnp.transpose(W)
    return _sc_gather(table, idx)
